# baseline (device time: 23930 ns/iter reference)
import jax
import jax.numpy as jnp
from jax import lax
from jax.experimental import pallas as pl
from jax.experimental.pallas import tpu as pltpu

N_DEV = 16
EPS = 1e-5
C_GLOBAL = 8192.0


def kernel(x, t_emb, W_scale, W_shift):
    b, s, c = x.shape

    def body(x_ref, t_ref, ws_ref, wh_ref, out_ref, comm_ref,
             send_sems, recv_sems):
        my = lax.axis_index("i")

        xv = x_ref[...]
        s1 = jnp.sum(xv, axis=-1)
        s2 = jnp.sum(xv * xv, axis=-1)
        part = jnp.concatenate([s1, s2], axis=0)
        comm_ref[my] = part

        barrier = pltpu.get_barrier_semaphore()
        for d in range(1, N_DEV):
            peer = lax.rem(my + d, N_DEV)
            pl.semaphore_signal(
                barrier, inc=1,
                device_id=(peer,), device_id_type=pl.DeviceIdType.MESH,
            )
        pl.semaphore_wait(barrier, N_DEV - 1)

        sends = []
        for d in range(1, N_DEV):
            peer = lax.rem(my + d, N_DEV)
            rdma = pltpu.make_async_remote_copy(
                src_ref=comm_ref.at[my],
                dst_ref=comm_ref.at[my],
                send_sem=send_sems.at[d],
                recv_sem=recv_sems.at[my],
                device_id=(peer,),
                device_id_type=pl.DeviceIdType.MESH,
            )
            rdma.start()
            sends.append(rdma)

        scale = jnp.dot(t_ref[...], ws_ref[...],
                        preferred_element_type=jnp.float32)
        shift = jnp.dot(t_ref[...], wh_ref[...],
                        preferred_element_type=jnp.float32)

        for d in range(1, N_DEV):
            src = lax.rem(my + N_DEV - d, N_DEV)
            recv = pltpu.make_async_remote_copy(
                src_ref=comm_ref.at[my],
                dst_ref=comm_ref.at[src],
                send_sem=send_sems.at[0],
                recv_sem=recv_sems.at[src],
                device_id=(src,),
                device_id_type=pl.DeviceIdType.MESH,
            )
            recv.wait_recv()
        for rdma in sends:
            rdma.wait_send()

        tot = jnp.sum(comm_ref[...], axis=0)
        mean = tot[0:2] / C_GLOBAL
        var = tot[2:4] / C_GLOBAL - mean * mean
        rstd = lax.rsqrt(var + EPS)
        h = (xv - mean[:, :, None]) * rstd[:, :, None]
        out_ref[...] = h * (1.0 + scale[:, None, :]) + shift[:, None, :]

    return pl.pallas_call(
        body,
        out_shape=jax.ShapeDtypeStruct((b, s, c), jnp.float32),
        in_specs=[pl.BlockSpec(memory_space=pltpu.VMEM)] * 4,
        out_specs=pl.BlockSpec(memory_space=pltpu.VMEM),
        scratch_shapes=[
            pltpu.VMEM((N_DEV, 4, s), jnp.float32),
            pltpu.SemaphoreType.DMA((N_DEV,)),
            pltpu.SemaphoreType.DMA((N_DEV,)),
        ],
        compiler_params=pltpu.CompilerParams(collective_id=0),
    )(x, t_emb, W_scale, W_shift)


# device time: 23024 ns/iter; 1.0394x vs baseline; 1.0394x over previous
import jax
import jax.numpy as jnp
from jax import lax
from jax.experimental import pallas as pl
from jax.experimental.pallas import tpu as pltpu

N_DEV = 16
PLANE = 4
EPS = 1e-5
C_GLOBAL = 8192.0


def kernel(x, t_emb, W_scale, W_shift):
    b, s, c = x.shape
    sh = s // 2

    def body(x_hbm, t_ref, ws_ref, wh_ref, out_ref,
             xv_ref, comm1_ref, comm2_ref,
             in_sem, zsem, p1_send, p1_recv, p2_send, p2_recv):
        my = lax.axis_index("i")
        zz = my // PLANE
        jj = lax.rem(my, PLANE)

        xcp = pltpu.make_async_copy(x_hbm, xv_ref, in_sem)
        xcp.start()

        barrier = pltpu.get_barrier_semaphore()
        for d in range(1, PLANE):
            peer_p = zz * PLANE + lax.rem(jj + d, PLANE)
            pl.semaphore_signal(
                barrier, inc=1,
                device_id=(peer_p,), device_id_type=pl.DeviceIdType.MESH,
            )
            peer_z = lax.rem(zz + d, PLANE) * PLANE + jj
            pl.semaphore_signal(
                zsem, inc=1,
                device_id=(peer_z,), device_id_type=pl.DeviceIdType.MESH,
            )

        scale = jnp.dot(t_ref[...], ws_ref[...],
                        preferred_element_type=jnp.float32)
        shift = jnp.dot(t_ref[...], wh_ref[...],
                        preferred_element_type=jnp.float32)
        xcp.wait()
        xv = xv_ref[...]
        s1 = jnp.sum(xv, axis=-1)
        s2 = jnp.sum(xv * xv, axis=-1)
        comm1_ref[jj] = jnp.concatenate([s1, s2], axis=0)

        pl.semaphore_wait(barrier, PLANE - 1)
        sends = []
        for d in range(1, PLANE):
            peer = zz * PLANE + lax.rem(jj + d, PLANE)
            rdma = pltpu.make_async_remote_copy(
                src_ref=comm1_ref.at[jj],
                dst_ref=comm1_ref.at[jj],
                send_sem=p1_send.at[d],
                recv_sem=p1_recv.at[jj],
                device_id=(peer,),
                device_id_type=pl.DeviceIdType.MESH,
            )
            rdma.start()
            sends.append(rdma)
        for d in range(1, PLANE):
            src = lax.rem(jj + PLANE - d, PLANE)
            pltpu.make_async_remote_copy(
                src_ref=comm1_ref.at[jj],
                dst_ref=comm1_ref.at[src],
                send_sem=p1_send.at[0],
                recv_sem=p1_recv.at[src],
                device_id=(src,),
                device_id_type=pl.DeviceIdType.MESH,
            ).wait_recv()

        comm2_ref[zz] = jnp.sum(comm1_ref[...], axis=0)

        pl.semaphore_wait(zsem, PLANE - 1)
        for d in range(1, PLANE):
            peer = lax.rem(zz + d, PLANE) * PLANE + jj
            rdma = pltpu.make_async_remote_copy(
                src_ref=comm2_ref.at[zz],
                dst_ref=comm2_ref.at[zz],
                send_sem=p2_send.at[d],
                recv_sem=p2_recv.at[zz],
                device_id=(peer,),
                device_id_type=pl.DeviceIdType.MESH,
            )
            rdma.start()
            sends.append(rdma)
        for d in range(1, PLANE):
            src = lax.rem(zz + PLANE - d, PLANE)
            pltpu.make_async_remote_copy(
                src_ref=comm2_ref.at[zz],
                dst_ref=comm2_ref.at[src],
                send_sem=p2_send.at[0],
                recv_sem=p2_recv.at[src],
                device_id=(src,),
                device_id_type=pl.DeviceIdType.MESH,
            ).wait_recv()

        tot = jnp.sum(comm2_ref[...], axis=0)
        mean = tot[0:2] / C_GLOBAL
        var = tot[2:4] / C_GLOBAL - mean * mean
        rstd = lax.rsqrt(var + EPS)

        h = (xv - mean[:, :, None]) * rstd[:, :, None]
        out_ref[...] = h * (1.0 + scale[:, None, :]) + shift[:, None, :]

        for rdma in sends:
            rdma.wait_send()

    return pl.pallas_call(
        body,
        out_shape=jax.ShapeDtypeStruct((b, s, c), jnp.float32),
        in_specs=[
            pl.BlockSpec(memory_space=pl.ANY),
            pl.BlockSpec(memory_space=pltpu.VMEM),
            pl.BlockSpec(memory_space=pltpu.VMEM),
            pl.BlockSpec(memory_space=pltpu.VMEM),
        ],
        out_specs=pl.BlockSpec(memory_space=pltpu.VMEM),
        scratch_shapes=[
            pltpu.VMEM((b, s, c), jnp.float32),
            pltpu.VMEM((PLANE, 4, s), jnp.float32),
            pltpu.VMEM((PLANE, 4, s), jnp.float32),
            pltpu.SemaphoreType.DMA,
            pltpu.SemaphoreType.REGULAR,
            pltpu.SemaphoreType.DMA((PLANE,)),
            pltpu.SemaphoreType.DMA((PLANE,)),
            pltpu.SemaphoreType.DMA((PLANE,)),
            pltpu.SemaphoreType.DMA((PLANE,)),
        ],
        compiler_params=pltpu.CompilerParams(collective_id=0),
    )(x, t_emb, W_scale, W_shift)
